# baseline (speedup 1.0000x reference)
import functools

import jax
import jax.numpy as jnp
from jax import lax
from jax.experimental import pallas as pl
from jax.experimental.pallas import tpu as pltpu
from jax.experimental.pallas import tpu_sc as plsc

NUM_REL = 256
ES = 32
D = ES * ES

NC = 2
NS = 16
NW = NC * NS

CH = 40


def _sc_embedding_lookup(table, idx_flat):
    B = idx_flat.shape[0]
    assert B % NW == 0
    b_per_w = B // NW
    assert b_per_w % CH == 0
    n_chunks = b_per_w // CH

    mesh = plsc.VectorSubcoreMesh(
        core_axis_name="c", subcore_axis_name="s", num_cores=NC, num_subcores=NS
    )

    @functools.partial(
        pl.kernel,
        out_type=jax.ShapeDtypeStruct((B, D), jnp.float32),
        mesh=mesh,
        scratch_types=[
            pltpu.VMEM((b_per_w,), jnp.int32),
            pltpu.VMEM((CH, D), jnp.float32),
            pltpu.VMEM((CH, D), jnp.float32),
            pltpu.SemaphoreType.DMA,
            pltpu.SemaphoreType.DMA,
            pltpu.SemaphoreType.DMA,
            pltpu.SemaphoreType.DMA,
        ],
    )
    def run(table_hbm, idx_hbm, out_hbm, idx_v, rows0, rows1, g0, g1, s0, s1):
        wid = lax.axis_index("s") * NC + lax.axis_index("c")
        base = wid * b_per_w
        pltpu.sync_copy(idx_hbm.at[pl.ds(base, b_per_w)], idx_v)

        bufs = (rows0, rows1)
        gsems = (g0, g1)
        ssems = (s0, s1)

        def gather(i):
            b = i % 2
            return pltpu.async_copy(
                table_hbm.at[idx_v.at[pl.ds(i * CH, CH)]], bufs[b], gsems[b]
            )

        def scatter(i):
            b = i % 2
            return pltpu.async_copy(
                bufs[b], out_hbm.at[pl.ds(base + i * CH, CH)], ssems[b]
            )

        gat = [None] * n_chunks
        scat = [None] * n_chunks
        gat[0] = gather(0)
        for i in range(n_chunks):
            if i + 1 < n_chunks:
                if i >= 1:
                    scat[i - 1].wait()
                gat[i + 1] = gather(i + 1)
            gat[i].wait()
            scat[i] = scatter(i)
        if n_chunks >= 2:
            scat[n_chunks - 2].wait()
        scat[n_chunks - 1].wait()

    return run(table, idx_flat)


def kernel(triples, norm_vector_weight):
    B_batch, N, _ = triples.shape
    idx_flat = triples.astype(jnp.int32).reshape(-1)
    out = _sc_embedding_lookup(norm_vector_weight, idx_flat)
    return out.reshape(B_batch, N, 1, D)


# speedup vs baseline: 3.7309x; 3.7309x over previous
import functools

import jax
import jax.numpy as jnp
from jax import lax
from jax.experimental import pallas as pl
from jax.experimental.pallas import tpu as pltpu
from jax.experimental.pallas import tpu_sc as plsc

NUM_REL = 256
ES = 32
D = ES * ES

NC = 2
NS = 16
NW = NC * NS

CH = 40


def _sc_embedding_lookup(table, idx_flat, B_batch, N):
    B = idx_flat.shape[0]
    assert B % NW == 0
    b_per_w = B // NW
    assert b_per_w % CH == 0
    n_chunks = b_per_w // CH
    assert CH % N == 0
    rows_per_chunk = CH // N
    assert B_batch % NW == 0

    mesh = plsc.VectorSubcoreMesh(
        core_axis_name="c", subcore_axis_name="s", num_cores=NC, num_subcores=NS
    )

    @functools.partial(
        pl.kernel,
        out_type=jax.ShapeDtypeStruct((B_batch, N, 1, D), jnp.float32),
        mesh=mesh,
        compiler_params=pltpu.CompilerParams(use_tc_tiling_on_sc=False),
        scratch_types=[
            pltpu.VMEM((b_per_w,), jnp.int32),
            pltpu.VMEM((CH, D), jnp.float32),
            pltpu.VMEM((CH, D), jnp.float32),
            pltpu.SemaphoreType.DMA,
            pltpu.SemaphoreType.DMA,
            pltpu.SemaphoreType.DMA,
            pltpu.SemaphoreType.DMA,
        ],
    )
    def run(table_hbm, idx_hbm, out_hbm, idx_v, rows0, rows1, g0, g1, s0, s1):
        wid = lax.axis_index("s") * NC + lax.axis_index("c")
        base = wid * b_per_w
        bbase = wid * (B_batch // NW)
        pltpu.sync_copy(idx_hbm.at[pl.ds(base, b_per_w)], idx_v)

        bufs = (rows0, rows1)
        gsems = (g0, g1)
        ssems = (s0, s1)

        def gather(i):
            b = i % 2
            return pltpu.async_copy(
                table_hbm.at[idx_v.at[pl.ds(i * CH, CH)]], bufs[b], gsems[b]
            )

        class _Copies:
            def __init__(self, cps):
                self.cps = cps

            def wait(self):
                for cp in self.cps:
                    cp.wait()

        def scatter(i):
            b = i % 2
            cps = [
                pltpu.async_copy(
                    bufs[b].at[pl.ds(j * N, N)],
                    out_hbm.at[bbase + i * rows_per_chunk + j, :, 0],
                    ssems[b],
                )
                for j in range(rows_per_chunk)
            ]
            return _Copies(cps)

        gat = [None] * n_chunks
        scat = [None] * n_chunks
        gat[0] = gather(0)
        for i in range(n_chunks):
            if i + 1 < n_chunks:
                if i >= 1:
                    scat[i - 1].wait()
                gat[i + 1] = gather(i + 1)
            gat[i].wait()
            scat[i] = scatter(i)
        if n_chunks >= 2:
            scat[n_chunks - 2].wait()
        scat[n_chunks - 1].wait()

    return run(table, idx_flat)


def kernel(triples, norm_vector_weight):
    B_batch, N, _ = triples.shape
    idx_flat = triples.astype(jnp.int32).reshape(-1)
    return _sc_embedding_lookup(norm_vector_weight, idx_flat, B_batch, N)


# speedup vs baseline: 6.3507x; 1.7022x over previous
import functools

import jax
import jax.numpy as jnp
from jax import lax
from jax.experimental import pallas as pl
from jax.experimental.pallas import tpu as pltpu
from jax.experimental.pallas import tpu_sc as plsc

NUM_REL = 256
ES = 32
D = ES * ES

NC = 2
NS = 16
NW = NC * NS

CH = 40


def _sc_embedding_lookup(table, idx_flat, B_batch, N):
    B = idx_flat.shape[0]
    assert B % NW == 0
    b_per_w = B // NW
    assert b_per_w % CH == 0
    n_chunks = b_per_w // CH
    assert CH % N == 0
    rows_per_chunk = CH // N
    assert B_batch % NW == 0

    mesh = plsc.VectorSubcoreMesh(
        core_axis_name="c", subcore_axis_name="s", num_cores=NC, num_subcores=NS
    )

    @functools.partial(
        pl.kernel,
        out_type=jax.ShapeDtypeStruct((B_batch, N, 1, D), jnp.float32),
        mesh=mesh,
        compiler_params=pltpu.CompilerParams(use_tc_tiling_on_sc=False),
        scratch_types=[
            pltpu.VMEM((b_per_w,), jnp.int32),
            pltpu.VMEM((CH, D), jnp.float32),
            pltpu.VMEM((CH, D), jnp.float32),
            pltpu.VMEM_SHARED((NUM_REL, D), jnp.float32),
            pltpu.SemaphoreType.DMA,
            pltpu.SemaphoreType.DMA,
            pltpu.SemaphoreType.DMA,
            pltpu.SemaphoreType.DMA,
        ],
    )
    def run(
        table_hbm, idx_hbm, out_hbm, idx_v, rows0, rows1, table_sp, g0, g1, s0, s1
    ):
        wid = lax.axis_index("s") * NC + lax.axis_index("c")
        base = wid * b_per_w
        bbase = wid * (B_batch // NW)
        sid = lax.axis_index("s")
        tchunk = NUM_REL // NS
        pltpu.sync_copy(
            table_hbm.at[pl.ds(sid * tchunk, tchunk)],
            table_sp.at[pl.ds(sid * tchunk, tchunk)],
        )
        pltpu.sync_copy(idx_hbm.at[pl.ds(base, b_per_w)], idx_v)
        plsc.subcore_barrier()

        bufs = (rows0, rows1)
        gsems = (g0, g1)
        ssems = (s0, s1)

        def gather(i):
            b = i % 2
            return pltpu.async_copy(
                table_sp.at[idx_v.at[pl.ds(i * CH, CH)]], bufs[b], gsems[b]
            )

        class _Copies:
            def __init__(self, cps):
                self.cps = cps

            def wait(self):
                for cp in self.cps:
                    cp.wait()

        def scatter(i):
            b = i % 2
            cps = [
                pltpu.async_copy(
                    bufs[b].at[pl.ds(j * N, N)],
                    out_hbm.at[bbase + i * rows_per_chunk + j, :, 0],
                    ssems[b],
                )
                for j in range(rows_per_chunk)
            ]
            return _Copies(cps)

        gat = [None] * n_chunks
        scat = [None] * n_chunks
        gat[0] = gather(0)
        for i in range(n_chunks):
            if i + 1 < n_chunks:
                if i >= 1:
                    scat[i - 1].wait()
                gat[i + 1] = gather(i + 1)
            gat[i].wait()
            scat[i] = scatter(i)
        if n_chunks >= 2:
            scat[n_chunks - 2].wait()
        scat[n_chunks - 1].wait()

    return run(table, idx_flat)


def kernel(triples, norm_vector_weight):
    B_batch, N, _ = triples.shape
    idx_flat = triples.astype(jnp.int32).reshape(-1)
    return _sc_embedding_lookup(norm_vector_weight, idx_flat, B_batch, N)


# speedup vs baseline: 6.4093x; 1.0092x over previous
import functools

import jax
import jax.numpy as jnp
from jax import lax
from jax.experimental import pallas as pl
from jax.experimental.pallas import tpu as pltpu
from jax.experimental.pallas import tpu_sc as plsc

NUM_REL = 256
ES = 32
D = ES * ES

NC = 2
NS = 16
NW = NC * NS

RPC = 2
NBUF = 2


def _sc_embedding_lookup(table, idx, B_batch, N):
    assert idx.shape == (B_batch, N)
    assert B_batch % (NW * RPC) == 0
    rows_w = B_batch // NW
    n_chunks = rows_w // RPC

    mesh = plsc.VectorSubcoreMesh(
        core_axis_name="c", subcore_axis_name="s", num_cores=NC, num_subcores=NS
    )

    @functools.partial(
        pl.kernel,
        out_type=jax.ShapeDtypeStruct((B_batch, N, 1, D), jnp.float32),
        mesh=mesh,
        compiler_params=pltpu.CompilerParams(use_tc_tiling_on_sc=False),
        scratch_types=[
            pltpu.VMEM((rows_w, N), jnp.int32),
            pltpu.VMEM_SHARED((NUM_REL, D), jnp.float32),
        ]
        + [pltpu.VMEM((RPC, N, D), jnp.float32)] * NBUF
        + [pltpu.SemaphoreType.DMA] * NBUF
        + [pltpu.SemaphoreType.DMA] * NBUF,
    )
    def run(table_hbm, idx_hbm, out_hbm, idx_v, table_sp, *rest):
        bufs = rest[:NBUF]
        gsems = rest[NBUF : 2 * NBUF]
        ssems = rest[2 * NBUF :]

        wid = lax.axis_index("s") * NC + lax.axis_index("c")
        bbase = wid * rows_w
        sid = lax.axis_index("s")
        tchunk = NUM_REL // NS
        pltpu.sync_copy(
            table_hbm.at[pl.ds(sid * tchunk, tchunk)],
            table_sp.at[pl.ds(sid * tchunk, tchunk)],
        )
        pltpu.sync_copy(idx_hbm.at[pl.ds(bbase, rows_w)], idx_v)
        plsc.subcore_barrier()

        def gather(i):
            b = i % NBUF
            return [
                pltpu.async_copy(
                    table_sp.at[idx_v.at[i * RPC + j]], bufs[b].at[j], gsems[b]
                )
                for j in range(RPC)
            ]

        def scatter(i):
            b = i % NBUF
            return pltpu.async_copy(
                bufs[b], out_hbm.at[pl.ds(bbase + i * RPC, RPC), :, 0], ssems[b]
            )

        gat = [None] * n_chunks
        scat = [None] * n_chunks
        gat[0] = gather(0)
        for i in range(n_chunks):
            nxt = i + 1
            if nxt < n_chunks:
                if nxt - NBUF >= 0:
                    scat[nxt - NBUF].wait()
                gat[nxt] = gather(nxt)
            for cp in gat[i]:
                cp.wait()
            scat[i] = scatter(i)
        for i in range(max(0, n_chunks - NBUF), n_chunks):
            scat[i].wait()

    return run(table, idx)


def kernel(triples, norm_vector_weight):
    B_batch, N, _ = triples.shape
    idx = triples.astype(jnp.int32).reshape(B_batch, N)
    return _sc_embedding_lookup(norm_vector_weight, idx, B_batch, N)


# speedup vs baseline: 6.4499x; 1.0063x over previous
import functools

import jax
import jax.numpy as jnp
from jax import lax
from jax.experimental import pallas as pl
from jax.experimental.pallas import tpu as pltpu
from jax.experimental.pallas import tpu_sc as plsc

NUM_REL = 256
ES = 32
D = ES * ES

NC = 2
NS = 16
NW = NC * NS

RPC = 1
NBUF = 4


def _sc_embedding_lookup(table, idx, B_batch, N):
    assert idx.shape == (B_batch, N)
    assert B_batch % (NW * RPC) == 0
    rows_w = B_batch // NW
    n_chunks = rows_w // RPC

    mesh = plsc.VectorSubcoreMesh(
        core_axis_name="c", subcore_axis_name="s", num_cores=NC, num_subcores=NS
    )

    @functools.partial(
        pl.kernel,
        out_type=jax.ShapeDtypeStruct((B_batch, N, 1, D), jnp.float32),
        mesh=mesh,
        compiler_params=pltpu.CompilerParams(use_tc_tiling_on_sc=False),
        scratch_types=[
            pltpu.VMEM((rows_w, N), jnp.int32),
            pltpu.VMEM_SHARED((NUM_REL, D), jnp.float32),
        ]
        + [pltpu.VMEM((RPC, N, D), jnp.float32)] * NBUF
        + [pltpu.SemaphoreType.DMA] * NBUF
        + [pltpu.SemaphoreType.DMA] * NBUF,
    )
    def run(table_hbm, idx_hbm, out_hbm, idx_v, table_sp, *rest):
        bufs = rest[:NBUF]
        gsems = rest[NBUF : 2 * NBUF]
        ssems = rest[2 * NBUF :]

        wid = lax.axis_index("s") * NC + lax.axis_index("c")
        bbase = wid * rows_w
        sid = lax.axis_index("s")
        tchunk = NUM_REL // NS
        pltpu.sync_copy(
            table_hbm.at[pl.ds(sid * tchunk, tchunk)],
            table_sp.at[pl.ds(sid * tchunk, tchunk)],
        )
        pltpu.sync_copy(idx_hbm.at[pl.ds(bbase, rows_w)], idx_v)
        plsc.subcore_barrier()

        def gather(i):
            b = i % NBUF
            return [
                pltpu.async_copy(
                    table_sp.at[idx_v.at[i * RPC + j]], bufs[b].at[j], gsems[b]
                )
                for j in range(RPC)
            ]

        def scatter(i):
            b = i % NBUF
            return pltpu.async_copy(
                bufs[b], out_hbm.at[pl.ds(bbase + i * RPC, RPC), :, 0], ssems[b]
            )

        gat = [None] * n_chunks
        scat = [None] * n_chunks
        gat[0] = gather(0)
        for i in range(n_chunks):
            nxt = i + 1
            if nxt < n_chunks:
                if nxt - NBUF >= 0:
                    scat[nxt - NBUF].wait()
                gat[nxt] = gather(nxt)
            for cp in gat[i]:
                cp.wait()
            scat[i] = scatter(i)
        for i in range(max(0, n_chunks - NBUF), n_chunks):
            scat[i].wait()

    return run(table, idx)


def kernel(triples, norm_vector_weight):
    B_batch, N, _ = triples.shape
    idx = triples.astype(jnp.int32).reshape(B_batch, N)
    return _sc_embedding_lookup(norm_vector_weight, idx, B_batch, N)


# speedup vs baseline: 6.4613x; 1.0018x over previous
import functools

import jax
import jax.numpy as jnp
from jax import lax
from jax.experimental import pallas as pl
from jax.experimental.pallas import tpu as pltpu
from jax.experimental.pallas import tpu_sc as plsc

NUM_REL = 256
ES = 32
D = ES * ES

NC = 2
NS = 16
NW = NC * NS

RPC = 1
NBUF = 5


def _sc_embedding_lookup(table, idx, B_batch, N):
    assert idx.shape == (B_batch, N)
    assert B_batch % (NW * RPC) == 0
    rows_w = B_batch // NW
    n_chunks = rows_w // RPC

    mesh = plsc.VectorSubcoreMesh(
        core_axis_name="c", subcore_axis_name="s", num_cores=NC, num_subcores=NS
    )

    @functools.partial(
        pl.kernel,
        out_type=jax.ShapeDtypeStruct((B_batch, N, 1, D), jnp.float32),
        mesh=mesh,
        compiler_params=pltpu.CompilerParams(use_tc_tiling_on_sc=False),
        scratch_types=[
            pltpu.VMEM((rows_w, N), jnp.int32),
            pltpu.VMEM_SHARED((NUM_REL, D), jnp.float32),
        ]
        + [pltpu.VMEM((RPC, N, D), jnp.float32)] * NBUF
        + [pltpu.SemaphoreType.DMA] * NBUF
        + [pltpu.SemaphoreType.DMA] * NBUF,
    )
    def run(table_hbm, idx_hbm, out_hbm, idx_v, table_sp, *rest):
        bufs = rest[:NBUF]
        gsems = rest[NBUF : 2 * NBUF]
        ssems = rest[2 * NBUF :]

        wid = lax.axis_index("s") * NC + lax.axis_index("c")
        bbase = wid * rows_w
        sid = lax.axis_index("s")
        tchunk = NUM_REL // NS
        pltpu.sync_copy(
            table_hbm.at[pl.ds(sid * tchunk, tchunk)],
            table_sp.at[pl.ds(sid * tchunk, tchunk)],
        )
        pltpu.sync_copy(idx_hbm.at[pl.ds(bbase, rows_w)], idx_v)
        plsc.subcore_barrier()

        def gather(i):
            b = i % NBUF
            return [
                pltpu.async_copy(
                    table_sp.at[idx_v.at[i * RPC + j]], bufs[b].at[j], gsems[b]
                )
                for j in range(RPC)
            ]

        def scatter(i):
            b = i % NBUF
            return pltpu.async_copy(
                bufs[b], out_hbm.at[pl.ds(bbase + i * RPC, RPC), :, 0], ssems[b]
            )

        gat = [None] * n_chunks
        scat = [None] * n_chunks
        gat[0] = gather(0)
        for i in range(n_chunks):
            nxt = i + 1
            if nxt < n_chunks:
                if nxt - NBUF >= 0:
                    scat[nxt - NBUF].wait()
                gat[nxt] = gather(nxt)
            for cp in gat[i]:
                cp.wait()
            scat[i] = scatter(i)
        for i in range(max(0, n_chunks - NBUF), n_chunks):
            scat[i].wait()

    return run(table, idx)


def kernel(triples, norm_vector_weight):
    B_batch, N, _ = triples.shape
    idx = triples.astype(jnp.int32).reshape(B_batch, N)
    return _sc_embedding_lookup(norm_vector_weight, idx, B_batch, N)


# speedup vs baseline: 6.5270x; 1.0102x over previous
import functools

import jax
import jax.numpy as jnp
from jax import lax
from jax.experimental import pallas as pl
from jax.experimental.pallas import tpu as pltpu
from jax.experimental.pallas import tpu_sc as plsc

NUM_REL = 256
ES = 32
D = ES * ES

NC = 2
NS = 16
NW = NC * NS

RPC = 1
NBUF = 5


def _sc_embedding_lookup(table, idx, B_batch, N):
    assert idx.shape == (B_batch, N)
    assert B_batch % (NW * RPC) == 0
    rows_w = B_batch // NW
    n_chunks = rows_w // RPC

    mesh = plsc.VectorSubcoreMesh(
        core_axis_name="c", subcore_axis_name="s", num_cores=NC, num_subcores=NS
    )

    @functools.partial(
        pl.kernel,
        out_type=jax.ShapeDtypeStruct((B_batch, N, 1, D), jnp.float32),
        mesh=mesh,
        compiler_params=pltpu.CompilerParams(use_tc_tiling_on_sc=False),
        scratch_types=[
            pltpu.VMEM((rows_w, N), jnp.int32),
            pltpu.VMEM_SHARED((NUM_REL, D), jnp.float32),
        ]
        + [pltpu.VMEM((RPC, N, D), jnp.float32)] * NBUF
        + [pltpu.SemaphoreType.DMA] * NBUF
        + [pltpu.SemaphoreType.DMA] * NBUF,
    )
    def run(table_hbm, idx_hbm, out_hbm, idx_v, table_sp, *rest):
        bufs = rest[:NBUF]
        gsems = rest[NBUF : 2 * NBUF]
        ssems = rest[2 * NBUF :]

        wid = lax.axis_index("s") * NC + lax.axis_index("c")
        bbase = wid * rows_w
        sid = lax.axis_index("s")
        tchunk = NUM_REL // NS
        tcp = pltpu.async_copy(
            table_hbm.at[pl.ds(sid * tchunk, tchunk)],
            table_sp.at[pl.ds(sid * tchunk, tchunk)],
            gsems[0],
        )
        icp = pltpu.async_copy(idx_hbm.at[pl.ds(bbase, rows_w)], idx_v, gsems[1])
        tcp.wait()
        icp.wait()
        plsc.subcore_barrier()

        def gather(i):
            b = i % NBUF
            return [
                pltpu.async_copy(
                    table_sp.at[idx_v.at[i * RPC + j]], bufs[b].at[j], gsems[b]
                )
                for j in range(RPC)
            ]

        def scatter(i):
            b = i % NBUF
            return pltpu.async_copy(
                bufs[b], out_hbm.at[pl.ds(bbase + i * RPC, RPC), :, 0], ssems[b]
            )

        gat = [None] * n_chunks
        scat = [None] * n_chunks
        gat[0] = gather(0)
        for i in range(n_chunks):
            nxt = i + 1
            if nxt < n_chunks:
                if nxt - NBUF >= 0:
                    scat[nxt - NBUF].wait()
                gat[nxt] = gather(nxt)
            for cp in gat[i]:
                cp.wait()
            scat[i] = scatter(i)
        for i in range(max(0, n_chunks - NBUF), n_chunks):
            scat[i].wait()

    return run(table, idx)


def kernel(triples, norm_vector_weight):
    B_batch, N, _ = triples.shape
    idx = triples.astype(jnp.int32).reshape(B_batch, N)
    return _sc_embedding_lookup(norm_vector_weight, idx, B_batch, N)


# speedup vs baseline: 6.5298x; 1.0004x over previous
import functools

import jax
import jax.numpy as jnp
from jax import lax
from jax.experimental import pallas as pl
from jax.experimental.pallas import tpu as pltpu
from jax.experimental.pallas import tpu_sc as plsc

NUM_REL = 256
ES = 32
D = ES * ES

NC = 2
NS = 16
NW = NC * NS

RPC = 1
NBUF = 5


def _sc_embedding_lookup(table, idx, B_batch, N):
    assert idx.shape == (B_batch, N)
    assert B_batch % (NW * RPC) == 0
    rows_w = B_batch // NW
    n_chunks = rows_w // RPC

    mesh = plsc.VectorSubcoreMesh(
        core_axis_name="c", subcore_axis_name="s", num_cores=NC, num_subcores=NS
    )

    @functools.partial(
        pl.kernel,
        out_type=jax.ShapeDtypeStruct((B_batch, N, 1, D), jnp.float32),
        mesh=mesh,
        compiler_params=pltpu.CompilerParams(use_tc_tiling_on_sc=False),
        scratch_types=[
            pltpu.VMEM((rows_w, N), jnp.int32),
            pltpu.VMEM_SHARED((NUM_REL, D), jnp.float32),
        ]
        + [pltpu.VMEM((RPC, N, D), jnp.float32)] * NBUF
        + [pltpu.SemaphoreType.DMA] * NBUF
        + [pltpu.SemaphoreType.DMA] * NBUF,
    )
    def run(table_hbm, idx_hbm, out_hbm, idx_v, table_sp, *rest):
        bufs = rest[:NBUF]
        gsems = rest[NBUF : 2 * NBUF]
        ssems = rest[2 * NBUF :]

        wid = lax.axis_index("s") * NC + lax.axis_index("c")
        bbase = wid * rows_w
        sid = lax.axis_index("s")
        tchunk = NUM_REL // NS
        tcp = pltpu.async_copy(
            table_hbm.at[pl.ds(sid * tchunk, tchunk)],
            table_sp.at[pl.ds(sid * tchunk, tchunk)],
            gsems[0],
        )
        icp = pltpu.async_copy(idx_hbm.at[pl.ds(bbase, rows_w)], idx_v, gsems[1])
        tcp.wait()
        icp.wait()
        plsc.subcore_barrier()

        def gather(i):
            b = i % NBUF
            return [
                pltpu.async_copy(
                    table_sp.at[idx_v.at[i * RPC + j]], bufs[b].at[j], gsems[b]
                )
                for j in range(RPC)
            ]

        def scatter(i):
            b = i % NBUF
            return pltpu.async_copy(
                bufs[b], out_hbm.at[pl.ds(bbase + i * RPC, RPC), :, 0], ssems[b]
            )

        LEAD = 2
        gat = [None] * n_chunks
        scat = [None] * n_chunks
        for i in range(min(LEAD, n_chunks)):
            gat[i] = gather(i)
        for i in range(n_chunks):
            nxt = i + LEAD
            if nxt < n_chunks:
                if nxt - NBUF >= 0:
                    scat[nxt - NBUF].wait()
                gat[nxt] = gather(nxt)
            for cp in gat[i]:
                cp.wait()
            scat[i] = scatter(i)
        for i in range(max(0, n_chunks - NBUF), n_chunks):
            scat[i].wait()

    return run(table, idx)


def kernel(triples, norm_vector_weight):
    B_batch, N, _ = triples.shape
    idx = triples.astype(jnp.int32).reshape(B_batch, N)
    return _sc_embedding_lookup(norm_vector_weight, idx, B_batch, N)
